# baseline (speedup 1.0000x reference)
import functools

import jax
import jax.numpy as jnp
from jax import lax
from jax.experimental import pallas as pl
from jax.experimental.pallas import tpu as pltpu
from jax.experimental.pallas import tpu_sc as plsc

B = 10000
NFEAT = 128
NHID = 128
NCLASS = 40
N0 = 10
N1 = 5

R = 400
A = 4800
AB = A // R

NW = 32
OUT_F = A * N0 * NFEAT
FPW = OUT_F // NW
CH_ROWS = 125
CH_OUT = CH_ROWS * NFEAT
CH_IN = CH_OUT * N1
NCH = FPW // CH_OUT

_sc_mesh = plsc.VectorSubcoreMesh(core_axis_name="c", subcore_axis_name="s")


@functools.partial(
    pl.kernel,
    mesh=_sc_mesh,
    out_type=jax.ShapeDtypeStruct((OUT_F,), jnp.float32),
    scratch_types=[
        pltpu.VMEM((CH_IN,), jnp.float32),
        pltpu.VMEM((CH_OUT,), jnp.float32),
    ],
)
def _agg2_sc(x2_hbm, out_hbm, in_v, out_v):
    wid = lax.axis_index("s") * 2 + lax.axis_index("c")
    in_base = wid * (FPW * N1)
    out_base = wid * FPW

    def do_chunk(ci, carry):
        pltpu.sync_copy(x2_hbm.at[pl.ds(in_base + ci * CH_IN, CH_IN)], in_v)

        def do_row(rr, c2):
            ib = rr * (N1 * NFEAT)
            ob = rr * NFEAT
            for f in range(NFEAT // 16):
                o = 16 * f
                acc = (in_v[pl.ds(ib + o, 16)]
                       + in_v[pl.ds(ib + NFEAT + o, 16)]
                       + in_v[pl.ds(ib + 2 * NFEAT + o, 16)]
                       + in_v[pl.ds(ib + 3 * NFEAT + o, 16)]
                       + in_v[pl.ds(ib + 4 * NFEAT + o, 16)])
                out_v[pl.ds(ob + o, 16)] = acc * (1.0 / N1)
            return c2

        lax.fori_loop(0, CH_ROWS, do_row, 0)
        pltpu.sync_copy(out_v, out_hbm.at[pl.ds(out_base + ci * CH_OUT, CH_OUT)])
        return carry

    lax.fori_loop(0, NCH, do_chunk, 0)


def _pool10(x):
    rows = jax.lax.broadcasted_iota(jnp.int32, (R, N0 * R), 0)
    cols = jax.lax.broadcasted_iota(jnp.int32, (R, N0 * R), 1)
    P = jnp.where(cols // N0 == rows, 1.0 / N0, 0.0)
    return jnp.dot(P, x, preferred_element_type=jnp.float32)


def _finish(x0b, x1b, agg2, ws0_ref, wn0_ref, b0_ref, ws1_ref, wn1_ref,
            b1_ref, o_ref):
    ws0 = ws0_ref[...]
    wn0 = wn0_ref[...]
    b0 = b0_ref[...]
    h1 = jax.nn.relu(jnp.dot(x1b, ws0, preferred_element_type=jnp.float32)
                     + jnp.dot(agg2, wn0, preferred_element_type=jnp.float32)
                     + b0)
    agg1 = _pool10(x1b)
    aggh = _pool10(h1)
    h0 = jax.nn.relu(jnp.dot(x0b, ws0, preferred_element_type=jnp.float32)
                     + jnp.dot(agg1, wn0, preferred_element_type=jnp.float32)
                     + b0)
    out = (jnp.dot(h0, ws1_ref[...], preferred_element_type=jnp.float32)
           + jnp.dot(aggh, wn1_ref[...], preferred_element_type=jnp.float32)
           + b1_ref[...])
    m = jnp.max(out, axis=1, keepdims=True)
    s = out - m
    lse = jnp.log(jnp.sum(jnp.exp(s), axis=1, keepdims=True))
    o_ref[...] = s - lse


def _block_from_x2(x0_ref, x1_ref, x2r_ref, ws0_ref, wn0_ref, b0_ref,
                   ws1_ref, wn1_ref, b1_ref, o_ref):
    x2b = x2r_ref[...]
    agg2 = (x2b[:, 0:128] + x2b[:, 128:256] + x2b[:, 256:384]
            + x2b[:, 384:512] + x2b[:, 512:640]) * (1.0 / N1)
    _finish(x0_ref[...], x1_ref[...], agg2, ws0_ref, wn0_ref, b0_ref,
            ws1_ref, wn1_ref, b1_ref, o_ref)


def _block_from_agg2(x0_ref, x1_ref, agg2_ref, ws0_ref, wn0_ref, b0_ref,
                     ws1_ref, wn1_ref, b1_ref, o_ref):
    _finish(x0_ref[...], x1_ref[...], agg2_ref[...], ws0_ref, wn0_ref,
            b0_ref, ws1_ref, wn1_ref, b1_ref, o_ref)


_W_SPECS = [
    pl.BlockSpec((NFEAT, NHID), lambda i: (0, 0)),
    pl.BlockSpec((NFEAT, NHID), lambda i: (0, 0)),
    pl.BlockSpec((1, NHID), lambda i: (0, 0)),
    pl.BlockSpec((NHID, NCLASS), lambda i: (0, 0)),
    pl.BlockSpec((NHID, NCLASS), lambda i: (0, 0)),
    pl.BlockSpec((1, NCLASS), lambda i: (0, 0)),
]


@jax.jit
def _run(x0, x1, x2r, x2f, W_self0, W_neigh0, b0, W_self1, W_neigh1, b1):
    weights = (W_self0, W_neigh0, b0, W_self1, W_neigh1, b1)

    agg2_a = _agg2_sc(x2f).reshape(A * N0, NFEAT)

    out_b = pl.pallas_call(
        _block_from_x2,
        grid=((B - A) // R,),
        in_specs=[
            pl.BlockSpec((R, NFEAT), lambda i: (i + AB, 0)),
            pl.BlockSpec((N0 * R, NFEAT), lambda i: (i + AB, 0)),
            pl.BlockSpec((N0 * R, N1 * NFEAT), lambda i: (i + AB, 0)),
        ] + _W_SPECS,
        out_specs=pl.BlockSpec((R, NCLASS), lambda i: (i, 0)),
        out_shape=jax.ShapeDtypeStruct((B - A, NCLASS), jnp.float32),
        compiler_params=pltpu.CompilerParams(
            dimension_semantics=("parallel",),
        ),
    )(x0, x1, x2r, *weights)

    out_a = pl.pallas_call(
        _block_from_agg2,
        grid=(A // R,),
        in_specs=[
            pl.BlockSpec((R, NFEAT), lambda i: (i, 0)),
            pl.BlockSpec((N0 * R, NFEAT), lambda i: (i, 0)),
            pl.BlockSpec((N0 * R, NFEAT), lambda i: (i, 0)),
        ] + _W_SPECS,
        out_specs=pl.BlockSpec((R, NCLASS), lambda i: (i, 0)),
        out_shape=jax.ShapeDtypeStruct((A, NCLASS), jnp.float32),
        compiler_params=pltpu.CompilerParams(
            dimension_semantics=("parallel",),
        ),
    )(x0, x1, agg2_a, *weights)

    return jnp.concatenate([out_a, out_b], axis=0)


def kernel(x0, x1, x2, W_self0, W_neigh0, b0, W_self1, W_neigh1, b1):
    return _run(x0, x1, x2.reshape(B * N0, N1 * NFEAT), x2.reshape(-1),
                W_self0, W_neigh0, b0.reshape(1, NHID),
                W_self1, W_neigh1, b1.reshape(1, NCLASS))


# speedup vs baseline: 2.9456x; 2.9456x over previous
import functools

import jax
import jax.numpy as jnp
from jax import lax
from jax.experimental import pallas as pl
from jax.experimental.pallas import tpu as pltpu
from jax.experimental.pallas import tpu_sc as plsc

B = 10000
NFEAT = 128
NHID = 128
NCLASS = 40
N0 = 10
N1 = 5

R = 400
A = 1600
AB = A // R

NW = 32
OUT_F = A * N0 * NFEAT
FPW = OUT_F // NW
CH_ROWS = 25
CH_OUT = CH_ROWS * NFEAT
CH_IN = CH_OUT * N1
NCH = FPW // CH_OUT
assert A % R == 0 and OUT_F % NW == 0 and FPW % CH_OUT == 0
assert NCH % 2 == 0, "ring loop processes chunks in pairs"

_sc_mesh = plsc.VectorSubcoreMesh(core_axis_name="c", subcore_axis_name="s")


@functools.partial(
    pl.kernel,
    mesh=_sc_mesh,
    out_type=jax.ShapeDtypeStruct((OUT_F,), jnp.float32),
    scratch_types=[
        pltpu.VMEM((2, CH_IN), jnp.float32),
        pltpu.VMEM((2, CH_OUT), jnp.float32),
        pltpu.SemaphoreType.DMA,
        pltpu.SemaphoreType.DMA,
        pltpu.SemaphoreType.DMA,
        pltpu.SemaphoreType.DMA,
    ],
)
def _agg2_sc(x2_hbm, out_hbm, in_v, out_v, si0, si1, so0, so1):
    wid = lax.axis_index("s") * 2 + lax.axis_index("c")
    in_base = wid * (FPW * N1)
    out_base = wid * FPW

    def start_in(ci, b, sem):
        pltpu.make_async_copy(
            x2_hbm.at[pl.ds(in_base + ci * CH_IN, CH_IN)], in_v.at[b], sem
        ).start()

    def wait_in(ci, b, sem):
        pltpu.make_async_copy(
            x2_hbm.at[pl.ds(in_base + ci * CH_IN, CH_IN)], in_v.at[b], sem
        ).wait()

    def start_out(ci, b, sem):
        pltpu.make_async_copy(
            out_v.at[b], out_hbm.at[pl.ds(out_base + ci * CH_OUT, CH_OUT)], sem
        ).start()

    def wait_out(ci, b, sem):
        pltpu.make_async_copy(
            out_v.at[b], out_hbm.at[pl.ds(out_base + ci * CH_OUT, CH_OUT)], sem
        ).wait()

    def compute(b):
        def row(rr, c2):
            ib = rr * (N1 * NFEAT)
            ob = rr * NFEAT
            for f in range(NFEAT // 16):
                o = 16 * f
                acc = (in_v[b, pl.ds(ib + o, 16)]
                       + in_v[b, pl.ds(ib + NFEAT + o, 16)]
                       + in_v[b, pl.ds(ib + 2 * NFEAT + o, 16)]
                       + in_v[b, pl.ds(ib + 3 * NFEAT + o, 16)]
                       + in_v[b, pl.ds(ib + 4 * NFEAT + o, 16)])
                out_v[b, pl.ds(ob + o, 16)] = acc * (1.0 / N1)
            return c2

        lax.fori_loop(0, CH_ROWS, row, 0)

    start_in(0, 0, si0)

    def super_step(g, carry):
        c0 = 2 * g
        c1 = 2 * g + 1
        start_in(c1, 1, si1)
        wait_in(c0, 0, si0)

        @pl.when(g > 0)
        def _():
            wait_out(c0 - 2, 0, so0)

        compute(0)
        start_out(c0, 0, so0)

        @pl.when(g + 1 < NCH // 2)
        def _():
            start_in(c0 + 2, 0, si0)

        wait_in(c1, 1, si1)

        @pl.when(g > 0)
        def _():
            wait_out(c1 - 2, 1, so1)

        compute(1)
        start_out(c1, 1, so1)
        return carry

    lax.fori_loop(0, NCH // 2, super_step, 0)
    wait_out(NCH - 2, 0, so0)
    wait_out(NCH - 1, 1, so1)


def _pool10(x):
    return _pool(x, N0, 1.0 / N0)


def _finish(x0b, x1b, agg2, ws0_ref, wn0_ref, b0_ref, ws1_ref, wn1_ref,
            b1_ref, o_ref):
    ws0 = ws0_ref[...]
    wn0 = wn0_ref[...]
    b0 = b0_ref[...]
    h1 = jax.nn.relu(jnp.dot(x1b, ws0, preferred_element_type=jnp.float32)
                     + jnp.dot(agg2, wn0, preferred_element_type=jnp.float32)
                     + b0)
    agg1 = _pool10(x1b)
    aggh = _pool10(h1)
    h0 = jax.nn.relu(jnp.dot(x0b, ws0, preferred_element_type=jnp.float32)
                     + jnp.dot(agg1, wn0, preferred_element_type=jnp.float32)
                     + b0)
    out = (jnp.dot(h0, ws1_ref[...], preferred_element_type=jnp.float32)
           + jnp.dot(aggh, wn1_ref[...], preferred_element_type=jnp.float32)
           + b1_ref[...])
    m = jnp.max(out, axis=1, keepdims=True)
    s = out - m
    lse = jnp.log(jnp.sum(jnp.exp(s), axis=1, keepdims=True))
    o_ref[...] = s - lse


def _pool(x, fan, scale):
    n = x.shape[0]
    gi = 8 * fan
    nb = n // gi
    x3 = x.reshape(nb, gi, NFEAT)
    r8 = jax.lax.broadcasted_iota(jnp.int32, (8, gi), 0)
    cg = jax.lax.broadcasted_iota(jnp.int32, (8, gi), 1)
    P = jnp.where(cg // fan == r8, scale, 0.0)
    Pb = jnp.broadcast_to(P, (nb, 8, gi))
    y = jax.lax.dot_general(Pb, x3, (((2,), (1,)), ((0,), (0,))),
                            preferred_element_type=jnp.float32)
    return y.reshape(n // fan, NFEAT)


def _block_from_x2(x0_ref, x1_ref, x2_ref, ws0_ref, wn0_ref, b0_ref,
                   ws1_ref, wn1_ref, b1_ref, o_ref):
    x2b = x2_ref[...]
    agg2 = _pool(x2b, N1, 1.0 / N1)
    _finish(x0_ref[...], x1_ref[...], agg2, ws0_ref, wn0_ref, b0_ref,
            ws1_ref, wn1_ref, b1_ref, o_ref)


def _block_from_agg2(x0_ref, x1_ref, agg2_ref, ws0_ref, wn0_ref, b0_ref,
                     ws1_ref, wn1_ref, b1_ref, o_ref):
    _finish(x0_ref[...], x1_ref[...], agg2_ref[...], ws0_ref, wn0_ref,
            b0_ref, ws1_ref, wn1_ref, b1_ref, o_ref)


_W_SPECS = [
    pl.BlockSpec((NFEAT, NHID), lambda i: (0, 0)),
    pl.BlockSpec((NFEAT, NHID), lambda i: (0, 0)),
    pl.BlockSpec((1, NHID), lambda i: (0, 0)),
    pl.BlockSpec((NHID, NCLASS), lambda i: (0, 0)),
    pl.BlockSpec((NHID, NCLASS), lambda i: (0, 0)),
    pl.BlockSpec((1, NCLASS), lambda i: (0, 0)),
]


@jax.jit
def _run(x0, x1, x2, x2f, W_self0, W_neigh0, b0, W_self1, W_neigh1, b1):
    weights = (W_self0, W_neigh0, b0, W_self1, W_neigh1, b1)

    agg2_a = _agg2_sc(x2f).reshape(A * N0, NFEAT)

    out_b = pl.pallas_call(
        _block_from_x2,
        grid=((B - A) // R,),
        in_specs=[
            pl.BlockSpec((R, NFEAT), lambda i: (i + AB, 0)),
            pl.BlockSpec((N0 * R, NFEAT), lambda i: (i + AB, 0)),
            pl.BlockSpec((N0 * N1 * R, NFEAT), lambda i: (i + AB, 0)),
        ] + _W_SPECS,
        out_specs=pl.BlockSpec((R, NCLASS), lambda i: (i, 0)),
        out_shape=jax.ShapeDtypeStruct((B - A, NCLASS), jnp.float32),
        compiler_params=pltpu.CompilerParams(
            dimension_semantics=("parallel",),
        ),
    )(x0, x1, x2, *weights)

    out_a = pl.pallas_call(
        _block_from_agg2,
        grid=(A // R,),
        in_specs=[
            pl.BlockSpec((R, NFEAT), lambda i: (i, 0)),
            pl.BlockSpec((N0 * R, NFEAT), lambda i: (i, 0)),
            pl.BlockSpec((N0 * R, NFEAT), lambda i: (i, 0)),
        ] + _W_SPECS,
        out_specs=pl.BlockSpec((R, NCLASS), lambda i: (i, 0)),
        out_shape=jax.ShapeDtypeStruct((A, NCLASS), jnp.float32),
        compiler_params=pltpu.CompilerParams(
            dimension_semantics=("parallel",),
        ),
    )(x0, x1, agg2_a, *weights)

    return jnp.concatenate([out_a, out_b], axis=0)


def kernel(x0, x1, x2, W_self0, W_neigh0, b0, W_self1, W_neigh1, b1):
    return _run(x0, x1, x2, x2.reshape(-1),
                W_self0, W_neigh0, b0.reshape(1, NHID),
                W_self1, W_neigh1, b1.reshape(1, NCLASS))


# speedup vs baseline: 3.0104x; 1.0220x over previous
import functools

import jax
import jax.numpy as jnp
from jax import lax
from jax.experimental import pallas as pl
from jax.experimental.pallas import tpu as pltpu
from jax.experimental.pallas import tpu_sc as plsc

B = 10000
NFEAT = 128
NHID = 128
NCLASS = 40
N0 = 10
N1 = 5

R = 400
A = 800
AB = A // R

NW = 32
OUT_F = A * N0 * NFEAT
FPW = OUT_F // NW
CH_ROWS = 25
CH_OUT = CH_ROWS * NFEAT
CH_IN = CH_OUT * N1
NCH = FPW // CH_OUT
assert A % R == 0 and OUT_F % NW == 0 and FPW % CH_OUT == 0
assert NCH % 2 == 0, "ring loop processes chunks in pairs"

_sc_mesh = plsc.VectorSubcoreMesh(core_axis_name="c", subcore_axis_name="s")


@functools.partial(
    pl.kernel,
    mesh=_sc_mesh,
    out_type=jax.ShapeDtypeStruct((OUT_F,), jnp.float32),
    scratch_types=[
        pltpu.VMEM((2, CH_IN), jnp.float32),
        pltpu.VMEM((2, CH_OUT), jnp.float32),
        pltpu.SemaphoreType.DMA,
        pltpu.SemaphoreType.DMA,
        pltpu.SemaphoreType.DMA,
        pltpu.SemaphoreType.DMA,
    ],
)
def _agg2_sc(x2_hbm, out_hbm, in_v, out_v, si0, si1, so0, so1):
    wid = lax.axis_index("s") * 2 + lax.axis_index("c")
    in_base = wid * (FPW * N1)
    out_base = wid * FPW

    def start_in(ci, b, sem):
        pltpu.make_async_copy(
            x2_hbm.at[pl.ds(in_base + ci * CH_IN, CH_IN)], in_v.at[b], sem
        ).start()

    def wait_in(ci, b, sem):
        pltpu.make_async_copy(
            x2_hbm.at[pl.ds(in_base + ci * CH_IN, CH_IN)], in_v.at[b], sem
        ).wait()

    def start_out(ci, b, sem):
        pltpu.make_async_copy(
            out_v.at[b], out_hbm.at[pl.ds(out_base + ci * CH_OUT, CH_OUT)], sem
        ).start()

    def wait_out(ci, b, sem):
        pltpu.make_async_copy(
            out_v.at[b], out_hbm.at[pl.ds(out_base + ci * CH_OUT, CH_OUT)], sem
        ).wait()

    def compute(b):
        def row(rr, c2):
            ib = rr * (N1 * NFEAT)
            ob = rr * NFEAT
            for f in range(NFEAT // 16):
                o = 16 * f
                acc = (in_v[b, pl.ds(ib + o, 16)]
                       + in_v[b, pl.ds(ib + NFEAT + o, 16)]
                       + in_v[b, pl.ds(ib + 2 * NFEAT + o, 16)]
                       + in_v[b, pl.ds(ib + 3 * NFEAT + o, 16)]
                       + in_v[b, pl.ds(ib + 4 * NFEAT + o, 16)])
                out_v[b, pl.ds(ob + o, 16)] = acc * (1.0 / N1)
            return c2

        lax.fori_loop(0, CH_ROWS, row, 0)

    start_in(0, 0, si0)

    def super_step(g, carry):
        c0 = 2 * g
        c1 = 2 * g + 1
        start_in(c1, 1, si1)
        wait_in(c0, 0, si0)

        @pl.when(g > 0)
        def _():
            wait_out(c0 - 2, 0, so0)

        compute(0)
        start_out(c0, 0, so0)

        @pl.when(g + 1 < NCH // 2)
        def _():
            start_in(c0 + 2, 0, si0)

        wait_in(c1, 1, si1)

        @pl.when(g > 0)
        def _():
            wait_out(c1 - 2, 1, so1)

        compute(1)
        start_out(c1, 1, so1)
        return carry

    lax.fori_loop(0, NCH // 2, super_step, 0)
    wait_out(NCH - 2, 0, so0)
    wait_out(NCH - 1, 1, so1)


def _pool10(x):
    return _pool(x, N0, 1.0 / N0)


def _finish(x0b, x1b, agg2, ws0_ref, wn0_ref, b0_ref, ws1_ref, wn1_ref,
            b1_ref, o_ref):
    ws0 = ws0_ref[...]
    wn0 = wn0_ref[...]
    b0 = b0_ref[...]
    h1 = jax.nn.relu(jnp.dot(x1b, ws0, preferred_element_type=jnp.float32)
                     + jnp.dot(agg2, wn0, preferred_element_type=jnp.float32)
                     + b0)
    agg1 = _pool10(x1b)
    aggh = _pool10(h1)
    h0 = jax.nn.relu(jnp.dot(x0b, ws0, preferred_element_type=jnp.float32)
                     + jnp.dot(agg1, wn0, preferred_element_type=jnp.float32)
                     + b0)
    out = (jnp.dot(h0, ws1_ref[...], preferred_element_type=jnp.float32)
           + jnp.dot(aggh, wn1_ref[...], preferred_element_type=jnp.float32)
           + b1_ref[...])
    m = jnp.max(out, axis=1, keepdims=True)
    s = out - m
    lse = jnp.log(jnp.sum(jnp.exp(s), axis=1, keepdims=True))
    o_ref[...] = s - lse


def _pool(x, fan, scale):
    n = x.shape[0]
    gi = 8 * fan
    nb = n // gi
    x3 = x.reshape(nb, gi, NFEAT)
    r8 = jax.lax.broadcasted_iota(jnp.int32, (8, gi), 0)
    cg = jax.lax.broadcasted_iota(jnp.int32, (8, gi), 1)
    P = jnp.where(cg // fan == r8, scale, 0.0)
    Pb = jnp.broadcast_to(P, (nb, 8, gi))
    y = jax.lax.dot_general(Pb, x3, (((2,), (1,)), ((0,), (0,))),
                            preferred_element_type=jnp.float32)
    return y.reshape(n // fan, NFEAT)


def _block_from_x2(x0_ref, x1_ref, x2_ref, ws0_ref, wn0_ref, b0_ref,
                   ws1_ref, wn1_ref, b1_ref, o_ref):
    x2b = x2_ref[...]
    agg2 = _pool(x2b, N1, 1.0 / N1)
    _finish(x0_ref[...], x1_ref[...], agg2, ws0_ref, wn0_ref, b0_ref,
            ws1_ref, wn1_ref, b1_ref, o_ref)


def _block_from_agg2(x0_ref, x1_ref, agg2_ref, ws0_ref, wn0_ref, b0_ref,
                     ws1_ref, wn1_ref, b1_ref, o_ref):
    _finish(x0_ref[...], x1_ref[...], agg2_ref[...], ws0_ref, wn0_ref,
            b0_ref, ws1_ref, wn1_ref, b1_ref, o_ref)


_W_SPECS = [
    pl.BlockSpec((NFEAT, NHID), lambda i: (0, 0)),
    pl.BlockSpec((NFEAT, NHID), lambda i: (0, 0)),
    pl.BlockSpec((1, NHID), lambda i: (0, 0)),
    pl.BlockSpec((NHID, NCLASS), lambda i: (0, 0)),
    pl.BlockSpec((NHID, NCLASS), lambda i: (0, 0)),
    pl.BlockSpec((1, NCLASS), lambda i: (0, 0)),
]


@jax.jit
def _run(x0, x1, x2, x2f, W_self0, W_neigh0, b0, W_self1, W_neigh1, b1):
    weights = (W_self0, W_neigh0, b0, W_self1, W_neigh1, b1)

    agg2_a = _agg2_sc(x2f).reshape(A * N0, NFEAT)

    out_b = pl.pallas_call(
        _block_from_x2,
        grid=((B - A) // R,),
        in_specs=[
            pl.BlockSpec((R, NFEAT), lambda i: (i + AB, 0)),
            pl.BlockSpec((N0 * R, NFEAT), lambda i: (i + AB, 0)),
            pl.BlockSpec((N0 * N1 * R, NFEAT), lambda i: (i + AB, 0)),
        ] + _W_SPECS,
        out_specs=pl.BlockSpec((R, NCLASS), lambda i: (i, 0)),
        out_shape=jax.ShapeDtypeStruct((B - A, NCLASS), jnp.float32),
        compiler_params=pltpu.CompilerParams(
            dimension_semantics=("parallel",),
        ),
    )(x0, x1, x2, *weights)

    out_a = pl.pallas_call(
        _block_from_agg2,
        grid=(A // R,),
        in_specs=[
            pl.BlockSpec((R, NFEAT), lambda i: (i, 0)),
            pl.BlockSpec((N0 * R, NFEAT), lambda i: (i, 0)),
            pl.BlockSpec((N0 * R, NFEAT), lambda i: (i, 0)),
        ] + _W_SPECS,
        out_specs=pl.BlockSpec((R, NCLASS), lambda i: (i, 0)),
        out_shape=jax.ShapeDtypeStruct((A, NCLASS), jnp.float32),
        compiler_params=pltpu.CompilerParams(
            dimension_semantics=("parallel",),
        ),
    )(x0, x1, agg2_a, *weights)

    return jnp.concatenate([out_a, out_b], axis=0)


def kernel(x0, x1, x2, W_self0, W_neigh0, b0, W_self1, W_neigh1, b1):
    return _run(x0, x1, x2, x2.reshape(-1),
                W_self0, W_neigh0, b0.reshape(1, NHID),
                W_self1, W_neigh1, b1.reshape(1, NCLASS))
